# baseline (speedup 1.0000x reference)
import functools

import jax
import jax.numpy as jnp
from jax import lax
from jax.experimental import pallas as pl
from jax.experimental.pallas import tpu as pltpu
from jax.experimental.pallas import tpu_sc as plsc

VOCAB = 100000
D = 64
DP = 128
BT = 16384
NC, NS = 2, 16
NW = NC * NS
BPW = BT // NW
CHUNK = 128
NCHUNK = BPW // CHUNK
NBUF = 3


@functools.cache
def _build_sc_gather():
    mesh = plsc.VectorSubcoreMesh(core_axis_name="c", subcore_axis_name="s")

    @functools.partial(
        pl.kernel,
        mesh=mesh,
        out_type=jax.ShapeDtypeStruct((BT, DP), jnp.float32),
        scratch_types=[
            pltpu.VMEM((NCHUNK, CHUNK), jnp.int32),
        ]
        + [pltpu.VMEM((CHUNK, DP), jnp.float32) for _ in range(NBUF)]
        + [pltpu.SemaphoreType.DMA, pltpu.SemaphoreType.DMA],
    )
    def _sc_gather(tab, idx, out, idx_v, *bufs_and_sems):
        bufs = bufs_and_sems[:NBUF]
        gsem, osem = bufs_and_sems[NBUF:]
        wid = lax.axis_index("s") * NC + lax.axis_index("c")
        pltpu.sync_copy(idx.at[pl.ds(wid * NCHUNK, NCHUNK)], idx_v)
        base = wid * BPW
        g = [None] * NCHUNK
        o = [None] * NCHUNK
        for j in range(min(NBUF, NCHUNK)):
            g[j] = pltpu.async_copy(tab.at[idx_v.at[j]], bufs[j % NBUF], gsem)
        for j in range(NCHUNK):
            dst = pl.ds(base + j * CHUNK, CHUNK)
            g[j].wait()
            o[j] = pltpu.async_copy(bufs[j % NBUF], out.at[dst], osem)
            nxt = j + NBUF
            if nxt < NCHUNK:
                o[j].wait()
                g[nxt] = pltpu.async_copy(tab.at[idx_v.at[nxt]], bufs[nxt % NBUF], gsem)
        for j in range(NCHUNK):
            if o[j] is not None and j + NBUF >= NCHUNK:
                o[j].wait()

    return _sc_gather


VB = 512
NPB = (VOCAB + VB - 1) // VB
PROWS = NPB * (VB // 2)


def _pack_body(t_ref, o_ref):
    t = t_ref[...]
    o_ref[:, :D] = t[:, : VB // 2].T
    o_ref[:, D:] = t[:, VB // 2 :].T


_pack = pl.pallas_call(
    _pack_body,
    grid=(NPB,),
    in_specs=[pl.BlockSpec((D, VB), lambda i: (0, i))],
    out_specs=pl.BlockSpec((VB // 2, DP), lambda i: (i, 0)),
    out_shape=jax.ShapeDtypeStruct((PROWS, DP), jnp.float32),
    compiler_params=pltpu.CompilerParams(dimension_semantics=("parallel",)),
)


def _prob_body(a_ref, b_ref, pa_ref, pb_ref, o_ref):
    ap = a_ref[...]
    bp = b_ref[...]
    pa = pa_ref[...][:, :, None] == 1
    pb = pb_ref[...][:, :, None] == 1
    a = jnp.where(pa, ap[:, :, D:], ap[:, :, :D])
    b = jnp.where(pb, bp[:, :, D:], bp[:, :, :D])
    dot = jnp.sum(a * b, axis=2)
    na = jnp.sqrt(jnp.sum(a * a, axis=2))
    nb = jnp.sqrt(jnp.sum(b * b, axis=2))
    denom = jnp.maximum(na * nb, 1e-8)
    o_ref[...] = (1.0 + dot / denom) * 0.5


RB = 16
_prob = pl.pallas_call(
    _prob_body,
    grid=(BT // (RB * 128),),
    in_specs=[
        pl.BlockSpec((RB, 128, DP), lambda i: (i, 0, 0)),
        pl.BlockSpec((RB, 128, DP), lambda i: (i, 0, 0)),
        pl.BlockSpec((RB, 128), lambda i: (i, 0)),
        pl.BlockSpec((RB, 128), lambda i: (i, 0)),
    ],
    out_specs=pl.BlockSpec((RB, 128), lambda i: (i, 0)),
    out_shape=jax.ShapeDtypeStruct((BT // 128, 128), jnp.float32),
)


def kernel(center_table, context_table, center, context):
    cp = _pack(center_table.T)
    xp = _pack(context_table.T)
    c32 = center.astype(jnp.int32)
    x32 = context.astype(jnp.int32)
    ci = (((c32 >> 9) << 8) | (c32 & 255)).reshape(NW * NCHUNK, CHUNK)
    xi = (((x32 >> 9) << 8) | (x32 & 255)).reshape(NW * NCHUNK, CHUNK)
    pa = ((c32 >> 8) & 1).reshape(BT // 128, 128)
    pb = ((x32 >> 8) & 1).reshape(BT // 128, 128)
    g = _build_sc_gather()
    a = g(cp, ci).reshape(BT // 128, 128, DP)
    b = g(xp, xi).reshape(BT // 128, 128, DP)
    return _prob(a, b, pa, pb).reshape(BT)


# speedup vs baseline: 1.7557x; 1.7557x over previous
import functools

import jax
import jax.numpy as jnp
from jax import lax
from jax.experimental import pallas as pl
from jax.experimental.pallas import tpu as pltpu
from jax.experimental.pallas import tpu_sc as plsc

VOCAB = 100000
D = 64
DP = 128
BT = 16384
NC, NS = 2, 16
NW = NC * NS
BPW = BT // NW
CHUNK = 128
NCHUNK = BPW // CHUNK
NBUF = 3


@functools.cache
def _build_sc_gather():
    mesh = plsc.VectorSubcoreMesh(core_axis_name="c", subcore_axis_name="s")

    @functools.partial(
        pl.kernel,
        mesh=mesh,
        out_type=jax.ShapeDtypeStruct((BT, DP), jnp.float32),
        scratch_types=[
            pltpu.VMEM((NCHUNK, CHUNK), jnp.int32),
        ]
        + [pltpu.VMEM((CHUNK, DP), jnp.float32) for _ in range(NBUF)]
        + [pltpu.SemaphoreType.DMA, pltpu.SemaphoreType.DMA],
    )
    def _sc_gather(tab, idx, out, idx_v, *bufs_and_sems):
        bufs = bufs_and_sems[:NBUF]
        gsem, osem = bufs_and_sems[NBUF:]
        wid = lax.axis_index("s") * NC + lax.axis_index("c")
        pltpu.sync_copy(idx.at[pl.ds(wid * NCHUNK, NCHUNK)], idx_v)
        base = wid * BPW
        g = [None] * NCHUNK
        o = [None] * NCHUNK
        for j in range(min(NBUF, NCHUNK)):
            g[j] = pltpu.async_copy(tab.at[idx_v.at[j]], bufs[j % NBUF], gsem)
        for j in range(NCHUNK):
            dst = pl.ds(base + j * CHUNK, CHUNK)
            g[j].wait()
            o[j] = pltpu.async_copy(bufs[j % NBUF], out.at[dst], osem)
            nxt = j + NBUF
            if nxt < NCHUNK:
                o[j].wait()
                g[nxt] = pltpu.async_copy(tab.at[idx_v.at[nxt]], bufs[nxt % NBUF], gsem)
        for j in range(NCHUNK):
            if o[j] is not None and j + NBUF >= NCHUNK:
                o[j].wait()

    return _sc_gather


def _prob_body(a_ref, b_ref, pa_ref, pb_ref, o_ref):
    ap = a_ref[...]
    bp = b_ref[...]
    pa = pa_ref[...][:, :, None] == 1
    pb = pb_ref[...][:, :, None] == 1
    a = jnp.where(pa, ap[:, :, D:], ap[:, :, :D])
    b = jnp.where(pb, bp[:, :, D:], bp[:, :, :D])
    dot = jnp.sum(a * b, axis=2)
    na = jnp.sqrt(jnp.sum(a * a, axis=2))
    nb = jnp.sqrt(jnp.sum(b * b, axis=2))
    denom = jnp.maximum(na * nb, 1e-8)
    o_ref[...] = (1.0 + dot / denom) * 0.5


RB = 16
_prob = pl.pallas_call(
    _prob_body,
    grid=(BT // (RB * 128),),
    in_specs=[
        pl.BlockSpec((RB, 128, DP), lambda i: (i, 0, 0)),
        pl.BlockSpec((RB, 128, DP), lambda i: (i, 0, 0)),
        pl.BlockSpec((RB, 128), lambda i: (i, 0)),
        pl.BlockSpec((RB, 128), lambda i: (i, 0)),
    ],
    out_specs=pl.BlockSpec((RB, 128), lambda i: (i, 0)),
    out_shape=jax.ShapeDtypeStruct((BT // 128, 128), jnp.float32),
)


def kernel(center_table, context_table, center, context):
    cp = center_table.reshape(VOCAB // 2, DP)
    xp = context_table.reshape(VOCAB // 2, DP)
    c32 = center.astype(jnp.int32)
    x32 = context.astype(jnp.int32)
    ci = (c32 >> 1).reshape(NW * NCHUNK, CHUNK)
    xi = (x32 >> 1).reshape(NW * NCHUNK, CHUNK)
    pa = (c32 & 1).reshape(BT // 128, 128)
    pb = (x32 & 1).reshape(BT // 128, 128)
    g = _build_sc_gather()
    a = g(cp, ci).reshape(BT // 128, 128, DP)
    b = g(xp, xi).reshape(BT // 128, 128, DP)
    return _prob(a, b, pa, pb).reshape(BT)
